# baseline (speedup 1.0000x reference)
import functools

import jax
import jax.numpy as jnp
from jax import lax
from jax.experimental import pallas as pl
from jax.experimental.pallas import tpu as pltpu
from jax.experimental.pallas import tpu_sc as plsc

_B = 16384
_DMF = 64
_DMLP = 128
_CH = 128
_NSETS = 3
_DEPTH = 4
_LAG_W = 2


@functools.lru_cache(maxsize=None)
def _make_gather(rows_a, rows_b):
    info = plsc.get_sparse_core_info()
    nc, ns = info.num_cores, info.num_subcores
    nw = nc * ns
    bpw = _B // nw
    nch = bpw // _CH
    nk = _NSETS * nch
    mesh = plsc.VectorSubcoreMesh(core_axis_name="c", subcore_axis_name="s")

    @functools.partial(
        pl.kernel,
        mesh=mesh,
        out_type=[jax.ShapeDtypeStruct((_B, _DMLP), jnp.float32)
                  for _ in range(_NSETS)],
        scratch_types=[
            pltpu.VMEM((nk, _CH), jnp.int32),
        ] + [pltpu.VMEM((_CH, _DMLP), jnp.float32) for _ in range(_DEPTH)] + [
            pltpu.SemaphoreType.DMA,
            pltpu.SemaphoreType.DMA,
        ],
    )
    def gather(idx_all, table_a, table_b, o_u, o_i, o_n,
               idx_v, *bufs_and_sems):
        bufs = bufs_and_sems[:_DEPTH]
        sem_g, sem_w = bufs_and_sems[_DEPTH], bufs_and_sems[_DEPTH + 1]
        tables = [table_a, table_b, table_b]
        outs = [o_u, o_i, o_n]
        wid = lax.axis_index("s") * nc + lax.axis_index("c")
        base = wid * bpw
        pltpu.sync_copy(idx_all.at[wid], idx_v)

        gcps = [None] * nk
        wcps = [None] * nk

        def issue_write(m):
            s, j = divmod(m, nch)
            gcps[m].wait()
            wcps[m] = pltpu.async_copy(
                bufs[m % _DEPTH],
                outs[s].at[pl.ds(base + j * _CH, _CH)], sem_w)

        for k in range(nk):
            if k >= _LAG_W:
                issue_write(k - _LAG_W)
            if k >= _DEPTH:
                wcps[k - _DEPTH].wait()
            s = k // nch
            gcps[k] = pltpu.async_copy(
                tables[s].at[idx_v.at[k]], bufs[k % _DEPTH], sem_g)
        for m in range(nk - _LAG_W, nk):
            issue_write(m)
        for m in range(nk - _DEPTH, nk):
            wcps[m].wait()

    return gather


def _repack_body(tu_r, ti_r, ou_r, oi_r):
    ou_r[...] = tu_r[...].T
    oi_r[...] = ti_r[...].T


def _repack(mf_user_t, mf_item_t):
    ni = mf_user_t.shape[1]
    bc = 4096
    nb = pl.cdiv(ni, bc)
    return pl.pallas_call(
        _repack_body,
        grid=(nb,),
        in_specs=[pl.BlockSpec((_DMF, bc), lambda i: (0, i)),
                  pl.BlockSpec((_DMF, bc), lambda i: (0, i))],
        out_specs=[pl.BlockSpec((bc, _DMF), lambda i: (i, 0)),
                   pl.BlockSpec((bc, _DMF), lambda i: (i, 0))],
        out_shape=[jax.ShapeDtypeStruct((ni, _DMF), jnp.float32),
                   jax.ShapeDtypeStruct((ni, _DMF), jnp.float32)],
        compiler_params=pltpu.CompilerParams(
            dimension_semantics=("arbitrary",)),
    )(mf_user_t, mf_item_t)


def _half(wide, par):
    sel = par == 1
    return jnp.where(sel, wide[:, _DMF:], wide[:, :_DMF])


def _tc_body(umf_r, imf_r, nmf_r, umlp_r, imlp_r, nmlp_r,
             pu_r, pi_r, pn_r,
             w1a_r, w1b_r, b1_r, w2_r, b2_r, w3_r, b3_r,
             wpmf_r, wpmlp_r, bp_r, out_r):
    w1a = w1a_r[...]
    w1b = w1b_r[...]
    b1 = b1_r[...]
    w2 = w2_r[...]
    b2 = b2_r[...]
    w3 = w3_r[...]
    b3 = b3_r[...]
    wpmf = wpmf_r[...]
    wpmlp = wpmlp_r[...]
    bp = bp_r[...]
    umf = _half(umf_r[...], pu_r[...])
    umlp = umlp_r[...]
    u1 = jnp.dot(umlp, w1a, preferred_element_type=jnp.float32)

    def score(imf, imlp):
        h = jnp.maximum(
            u1 + jnp.dot(imlp, w1b, preferred_element_type=jnp.float32) + b1,
            0.0)
        h = jnp.maximum(
            jnp.dot(h, w2, preferred_element_type=jnp.float32) + b2, 0.0)
        h = jnp.maximum(
            jnp.dot(h, w3, preferred_element_type=jnp.float32) + b3, 0.0)
        logit = (jnp.sum(umf * imf * wpmf, axis=1, keepdims=True)
                 + jnp.sum(h * wpmlp, axis=1, keepdims=True) + bp)
        return jax.nn.sigmoid(logit)

    ps = score(_half(imf_r[...], pi_r[...]), imlp_r[...])
    ns = score(_half(nmf_r[...], pn_r[...]), nmlp_r[...])
    part = jnp.sum(jax.nn.softplus(ns - ps)) * (1.0 / _B)

    @pl.when(pl.program_id(0) == 0)
    def _():
        out_r[...] = jnp.zeros_like(out_r)

    out_r[...] += part


def _tc_loss(umf, imf, nmf, umlp, imlp, nmlp, pu, pi, pn,
             w1a, w1b, b1, w2, b2, w3, b3, wpmf, wpmlp, bp, *,
             interpret=False):
    bb = 2048
    nb = _B // bb

    def fixed(shape):
        return pl.BlockSpec(shape, lambda i: (0, 0))

    def batched(d):
        return pl.BlockSpec((bb, d), lambda i: (i, 0))

    return pl.pallas_call(
        _tc_body,
        grid=(nb,),
        in_specs=[
            batched(_DMLP), batched(_DMLP), batched(_DMLP),
            batched(_DMLP), batched(_DMLP), batched(_DMLP),
            batched(1), batched(1), batched(1),
            fixed((_DMLP, _DMLP)), fixed((_DMLP, _DMLP)), fixed((1, _DMLP)),
            fixed((_DMLP, 64)), fixed((1, 64)),
            fixed((64, 32)), fixed((1, 32)),
            fixed((1, _DMF)), fixed((1, 32)), fixed((1, 1)),
        ],
        out_specs=pl.BlockSpec((1, 1), lambda i: (0, 0)),
        out_shape=jax.ShapeDtypeStruct((1, 1), jnp.float32),
        compiler_params=pltpu.CompilerParams(
            dimension_semantics=("arbitrary",)),
        interpret=interpret,
    )(umf, imf, nmf, umlp, imlp, nmlp, pu, pi, pn,
      w1a, w1b, b1, w2, b2, w3, b3, wpmf, wpmlp, bp)


def _chunked(stack):
    nw = 32
    bpw = _B // nw
    return (stack.reshape(_NSETS, nw, bpw // _CH, _CH)
            .transpose(1, 0, 2, 3)
            .reshape(nw, _NSETS * (bpw // _CH), _CH))


def kernel(user, item, neg_item, mf_user, mf_item, mlp_user, mlp_item,
           W1, b1, W2, b2, W3, b3, Wp, bp):
    user = user.astype(jnp.int32)
    item = item.astype(jnp.int32)
    neg_item = neg_item.astype(jnp.int32)
    idx_mlp = _chunked(jnp.stack([user, item, neg_item]))
    idx_mf = _chunked(jnp.stack([user >> 1, item >> 1, neg_item >> 1]))
    pu = (user & 1).reshape(_B, 1)
    pi = (item & 1).reshape(_B, 1)
    pn = (neg_item & 1).reshape(_B, 1)

    umlp, imlp, nmlp = _make_gather(mlp_user.shape[0], mlp_item.shape[0])(
        idx_mlp, mlp_user, mlp_item)

    mf_user_r, mf_item_r = _repack(mf_user.T, mf_item.T)
    mf_user2 = mf_user_r.reshape(-1, _DMLP)
    mf_item2 = mf_item_r.reshape(-1, _DMLP)
    umf, imf, nmf = _make_gather(mf_user2.shape[0], mf_item2.shape[0])(
        idx_mf, mf_user2, mf_item2)

    w1a = W1[:_DMLP]
    w1b = W1[_DMLP:]
    wp = Wp.reshape(1, _DMF + 32)
    out = _tc_loss(
        umf, imf, nmf, umlp, imlp, nmlp, pu, pi, pn,
        w1a, w1b, b1.reshape(1, _DMLP), W2, b2.reshape(1, 64),
        W3, b3.reshape(1, 32), wp[:, :_DMF], wp[:, _DMF:], bp.reshape(1, 1))
    return out[0, 0]


# speedup vs baseline: 1.2800x; 1.2800x over previous
import functools

import jax
import jax.numpy as jnp
from jax import lax
from jax.experimental import pallas as pl
from jax.experimental.pallas import tpu as pltpu
from jax.experimental.pallas import tpu_sc as plsc

_B = 16384
_DMF = 64
_DMLP = 128
_CH = 128
_NSETS = 3
_DEPTH = 4
_LAG_W = 2


@functools.lru_cache(maxsize=None)
def _make_gather(rows_a, rows_b):
    info = plsc.get_sparse_core_info()
    nc, ns = info.num_cores, info.num_subcores
    nw = nc * ns
    bpw = _B // nw
    nch = bpw // _CH
    nk = _NSETS * nch
    mesh = plsc.VectorSubcoreMesh(core_axis_name="c", subcore_axis_name="s")

    @functools.partial(
        pl.kernel,
        mesh=mesh,
        out_type=[jax.ShapeDtypeStruct((_B, _DMLP), jnp.float32)
                  for _ in range(_NSETS)],
        scratch_types=[
            pltpu.VMEM((nk, _CH), jnp.int32),
        ] + [pltpu.VMEM((_CH, _DMLP), jnp.float32) for _ in range(_DEPTH)] + [
            pltpu.SemaphoreType.DMA,
            pltpu.SemaphoreType.DMA,
        ],
    )
    def gather(idx_all, table_a, table_b, o_u, o_i, o_n,
               idx_v, *bufs_and_sems):
        bufs = bufs_and_sems[:_DEPTH]
        sem_g, sem_w = bufs_and_sems[_DEPTH], bufs_and_sems[_DEPTH + 1]
        tables = [table_a, table_b, table_b]
        outs = [o_u, o_i, o_n]
        wid = lax.axis_index("s") * nc + lax.axis_index("c")
        base = wid * bpw
        pltpu.sync_copy(idx_all.at[wid], idx_v)

        gcps = [None] * nk
        wcps = [None] * nk

        def issue_write(m):
            s, j = divmod(m, nch)
            gcps[m].wait()
            wcps[m] = pltpu.async_copy(
                bufs[m % _DEPTH],
                outs[s].at[pl.ds(base + j * _CH, _CH)], sem_w)

        for k in range(nk):
            if k >= _LAG_W:
                issue_write(k - _LAG_W)
            if k >= _DEPTH:
                wcps[k - _DEPTH].wait()
            s = k // nch
            gcps[k] = pltpu.async_copy(
                tables[s].at[idx_v.at[k]], bufs[k % _DEPTH], sem_g)
        for m in range(nk - _LAG_W, nk):
            issue_write(m)
        for m in range(nk - _DEPTH, nk):
            wcps[m].wait()

    return gather


_RB = 2048
_RNB = 25
_OFF = _RB * _RNB


def _repack_body(au_r, bu_r, ai_r, bi_r, ou_r, oi_r):
    ou_r[:, :_DMF] = au_r[...].T
    ou_r[:, _DMF:] = bu_r[...].T
    oi_r[:, :_DMF] = ai_r[...].T
    oi_r[:, _DMF:] = bi_r[...].T


def _repack(mf_user_t, mf_item_t):
    pad = 2 * _OFF - mf_user_t.shape[1]
    mf_user_t = jnp.pad(mf_user_t, ((0, 0), (0, pad)))
    mf_item_t = jnp.pad(mf_item_t, ((0, 0), (0, pad)))
    return pl.pallas_call(
        _repack_body,
        grid=(_RNB,),
        in_specs=[pl.BlockSpec((_DMF, _RB), lambda i: (0, i)),
                  pl.BlockSpec((_DMF, _RB), lambda i: (0, i + _RNB)),
                  pl.BlockSpec((_DMF, _RB), lambda i: (0, i)),
                  pl.BlockSpec((_DMF, _RB), lambda i: (0, i + _RNB))],
        out_specs=[pl.BlockSpec((_RB, _DMLP), lambda i: (i, 0)),
                   pl.BlockSpec((_RB, _DMLP), lambda i: (i, 0))],
        out_shape=[jax.ShapeDtypeStruct((_OFF, _DMLP), jnp.float32),
                   jax.ShapeDtypeStruct((_OFF, _DMLP), jnp.float32)],
        compiler_params=pltpu.CompilerParams(
            dimension_semantics=("arbitrary",)),
    )(mf_user_t, mf_user_t, mf_item_t, mf_item_t)


def _half(wide, par):
    sel = par == 1
    return jnp.where(sel, wide[:, _DMF:], wide[:, :_DMF])


def _tc_body(umf_r, imf_r, nmf_r, umlp_r, imlp_r, nmlp_r,
             pu_r, pi_r, pn_r,
             w1a_r, w1b_r, b1_r, w2_r, b2_r, w3_r, b3_r,
             wpmf_r, wpmlp_r, bp_r, out_r):
    w1a = w1a_r[...]
    w1b = w1b_r[...]
    b1 = b1_r[...]
    w2 = w2_r[...]
    b2 = b2_r[...]
    w3 = w3_r[...]
    b3 = b3_r[...]
    wpmf = wpmf_r[...]
    wpmlp = wpmlp_r[...]
    bp = bp_r[...]
    umf = _half(umf_r[...], pu_r[...])
    umlp = umlp_r[...]
    u1 = jnp.dot(umlp, w1a, preferred_element_type=jnp.float32)

    def score(imf, imlp):
        h = jnp.maximum(
            u1 + jnp.dot(imlp, w1b, preferred_element_type=jnp.float32) + b1,
            0.0)
        h = jnp.maximum(
            jnp.dot(h, w2, preferred_element_type=jnp.float32) + b2, 0.0)
        h = jnp.maximum(
            jnp.dot(h, w3, preferred_element_type=jnp.float32) + b3, 0.0)
        logit = (jnp.sum(umf * imf * wpmf, axis=1, keepdims=True)
                 + jnp.sum(h * wpmlp, axis=1, keepdims=True) + bp)
        return jax.nn.sigmoid(logit)

    ps = score(_half(imf_r[...], pi_r[...]), imlp_r[...])
    ns = score(_half(nmf_r[...], pn_r[...]), nmlp_r[...])
    part = jnp.sum(jax.nn.softplus(ns - ps)) * (1.0 / _B)

    @pl.when(pl.program_id(0) == 0)
    def _():
        out_r[...] = jnp.zeros_like(out_r)

    out_r[...] += part


def _tc_loss(umf, imf, nmf, umlp, imlp, nmlp, pu, pi, pn,
             w1a, w1b, b1, w2, b2, w3, b3, wpmf, wpmlp, bp, *,
             interpret=False):
    bb = 2048
    nb = _B // bb

    def fixed(shape):
        return pl.BlockSpec(shape, lambda i: (0, 0))

    def batched(d):
        return pl.BlockSpec((bb, d), lambda i: (i, 0))

    return pl.pallas_call(
        _tc_body,
        grid=(nb,),
        in_specs=[
            batched(_DMLP), batched(_DMLP), batched(_DMLP),
            batched(_DMLP), batched(_DMLP), batched(_DMLP),
            batched(1), batched(1), batched(1),
            fixed((_DMLP, _DMLP)), fixed((_DMLP, _DMLP)), fixed((1, _DMLP)),
            fixed((_DMLP, 64)), fixed((1, 64)),
            fixed((64, 32)), fixed((1, 32)),
            fixed((1, _DMF)), fixed((1, 32)), fixed((1, 1)),
        ],
        out_specs=pl.BlockSpec((1, 1), lambda i: (0, 0)),
        out_shape=jax.ShapeDtypeStruct((1, 1), jnp.float32),
        compiler_params=pltpu.CompilerParams(
            dimension_semantics=("arbitrary",)),
        interpret=interpret,
    )(umf, imf, nmf, umlp, imlp, nmlp, pu, pi, pn,
      w1a, w1b, b1, w2, b2, w3, b3, wpmf, wpmlp, bp)


def _chunked(stack):
    nw = 32
    bpw = _B // nw
    return (stack.reshape(_NSETS, nw, bpw // _CH, _CH)
            .transpose(1, 0, 2, 3)
            .reshape(nw, _NSETS * (bpw // _CH), _CH))


def kernel(user, item, neg_item, mf_user, mf_item, mlp_user, mlp_item,
           W1, b1, W2, b2, W3, b3, Wp, bp):
    user = user.astype(jnp.int32)
    item = item.astype(jnp.int32)
    neg_item = neg_item.astype(jnp.int32)
    hu = (user >= _OFF).astype(jnp.int32)
    hi = (item >= _OFF).astype(jnp.int32)
    hn = (neg_item >= _OFF).astype(jnp.int32)
    idx_mlp = _chunked(jnp.stack([user, item, neg_item]))
    idx_mf = _chunked(jnp.stack([user - hu * _OFF, item - hi * _OFF,
                                 neg_item - hn * _OFF]))
    pu = hu.reshape(_B, 1)
    pi = hi.reshape(_B, 1)
    pn = hn.reshape(_B, 1)

    umlp, imlp, nmlp = _make_gather(mlp_user.shape[0], mlp_item.shape[0])(
        idx_mlp, mlp_user, mlp_item)

    mf_user2, mf_item2 = _repack(mf_user.T, mf_item.T)
    umf, imf, nmf = _make_gather(mf_user2.shape[0], mf_item2.shape[0])(
        idx_mf, mf_user2, mf_item2)

    w1a = W1[:_DMLP]
    w1b = W1[_DMLP:]
    wp = Wp.reshape(1, _DMF + 32)
    out = _tc_loss(
        umf, imf, nmf, umlp, imlp, nmlp, pu, pi, pn,
        w1a, w1b, b1.reshape(1, _DMLP), W2, b2.reshape(1, 64),
        W3, b3.reshape(1, 32), wp[:, :_DMF], wp[:, _DMF:], bp.reshape(1, 1))
    return out[0, 0]


# speedup vs baseline: 1.4076x; 1.0997x over previous
import functools

import jax
import jax.numpy as jnp
from jax import lax
from jax.experimental import pallas as pl
from jax.experimental.pallas import tpu as pltpu
from jax.experimental.pallas import tpu_sc as plsc

_B = 16384
_DMF = 64
_DMLP = 128
_CH = 128
_NSETS = 3
_DEPTH = 4
_LAG_W = 2


@functools.lru_cache(maxsize=None)
def _make_gather(rows_a, rows_b):
    info = plsc.get_sparse_core_info()
    nc, ns = info.num_cores, info.num_subcores
    nw = nc * ns
    bpw = _B // nw
    nch = bpw // _CH
    nk = _NSETS * nch
    mesh = plsc.VectorSubcoreMesh(core_axis_name="c", subcore_axis_name="s")

    @functools.partial(
        pl.kernel,
        mesh=mesh,
        out_type=[jax.ShapeDtypeStruct((_B, _DMLP), jnp.float32)
                  for _ in range(_NSETS)],
        scratch_types=[
            pltpu.VMEM((nk, _CH), jnp.int32),
        ] + [pltpu.VMEM((_CH, _DMLP), jnp.float32) for _ in range(_DEPTH)] + [
            pltpu.SemaphoreType.DMA,
            pltpu.SemaphoreType.DMA,
        ],
    )
    def gather(idx_all, table_a, table_b, o_u, o_i, o_n,
               idx_v, *bufs_and_sems):
        bufs = bufs_and_sems[:_DEPTH]
        sem_g, sem_w = bufs_and_sems[_DEPTH], bufs_and_sems[_DEPTH + 1]
        tables = [table_a, table_b, table_b]
        outs = [o_u, o_i, o_n]
        wid = lax.axis_index("s") * nc + lax.axis_index("c")
        base = wid * bpw
        pltpu.sync_copy(idx_all.at[wid], idx_v)

        gcps = [None] * nk
        wcps = [None] * nk

        def issue_write(m):
            s, j = divmod(m, nch)
            gcps[m].wait()
            wcps[m] = pltpu.async_copy(
                bufs[m % _DEPTH],
                outs[s].at[pl.ds(base + j * _CH, _CH)], sem_w)

        for k in range(nk):
            if k >= _LAG_W:
                issue_write(k - _LAG_W)
            if k >= _DEPTH:
                wcps[k - _DEPTH].wait()
            s = k // nch
            gcps[k] = pltpu.async_copy(
                tables[s].at[idx_v.at[k]], bufs[k % _DEPTH], sem_g)
        for m in range(nk - _LAG_W, nk):
            issue_write(m)
        for m in range(nk - _DEPTH, nk):
            wcps[m].wait()

    return gather


_NI = 100000
_RB = 1024
_RNB = 50
_OFF = _RB * _RNB
_RNB_IN = (_NI - _OFF) // _RB
_TAIL0 = _OFF + _RNB_IN * _RB


def _repack_body(au_r, bu_r, tu_r, ai_r, bi_r, ti_r, ou_r, oi_r):
    i = pl.program_id(0)
    ou_r[:, :_DMF] = au_r[...].T
    oi_r[:, :_DMF] = ai_r[...].T

    @pl.when(i < _RNB_IN)
    def _():
        ou_r[:, _DMF:] = bu_r[...].T
        oi_r[:, _DMF:] = bi_r[...].T

    @pl.when(i >= _RNB_IN)
    def _():
        ou_r[:, _DMF:] = tu_r[...].T
        oi_r[:, _DMF:] = ti_r[...].T


def _repack(mf_user_t, mf_item_t):
    ntail = _RNB - _RNB_IN
    tw = ntail * _RB

    def tail(t):
        return jnp.pad(t[:, _TAIL0:], ((0, 0), (0, tw - (_NI - _TAIL0))))

    a_map = lambda i: (0, i)
    b_map = lambda i: (0, jnp.minimum(i + _RNB, _RNB + _RNB_IN - 1))
    t_map = lambda i: (0, jnp.clip(i - _RNB_IN, 0, ntail - 1))
    return pl.pallas_call(
        _repack_body,
        grid=(_RNB,),
        in_specs=[pl.BlockSpec((_DMF, _RB), a_map),
                  pl.BlockSpec((_DMF, _RB), b_map),
                  pl.BlockSpec((_DMF, _RB), t_map),
                  pl.BlockSpec((_DMF, _RB), a_map),
                  pl.BlockSpec((_DMF, _RB), b_map),
                  pl.BlockSpec((_DMF, _RB), t_map)],
        out_specs=[pl.BlockSpec((_RB, _DMLP), lambda i: (i, 0)),
                   pl.BlockSpec((_RB, _DMLP), lambda i: (i, 0))],
        out_shape=[jax.ShapeDtypeStruct((_OFF, _DMLP), jnp.float32),
                   jax.ShapeDtypeStruct((_OFF, _DMLP), jnp.float32)],
        compiler_params=pltpu.CompilerParams(
            dimension_semantics=("arbitrary",)),
    )(mf_user_t, mf_user_t, tail(mf_user_t),
      mf_item_t, mf_item_t, tail(mf_item_t))


def _half(wide, par):
    sel = par == 1
    return jnp.where(sel, wide[:, _DMF:], wide[:, :_DMF])


def _tc_body(umf_r, imf_r, nmf_r, umlp_r, imlp_r, nmlp_r,
             pu_r, pi_r, pn_r,
             w1a_r, w1b_r, b1_r, w2_r, b2_r, w3_r, b3_r,
             wpmf_r, wpmlp_r, bp_r, out_r):
    w1a = w1a_r[...]
    w1b = w1b_r[...]
    b1 = b1_r[...]
    w2 = w2_r[...]
    b2 = b2_r[...]
    w3 = w3_r[...]
    b3 = b3_r[...]
    wpmf = wpmf_r[...]
    wpmlp = wpmlp_r[...]
    bp = bp_r[...]
    umf = _half(umf_r[...], pu_r[...])
    umlp = umlp_r[...]
    u1 = jnp.dot(umlp, w1a, preferred_element_type=jnp.float32)

    def score(imf, imlp):
        h = jnp.maximum(
            u1 + jnp.dot(imlp, w1b, preferred_element_type=jnp.float32) + b1,
            0.0)
        h = jnp.maximum(
            jnp.dot(h, w2, preferred_element_type=jnp.float32) + b2, 0.0)
        h = jnp.maximum(
            jnp.dot(h, w3, preferred_element_type=jnp.float32) + b3, 0.0)
        logit = (jnp.dot(umf * imf, wpmf, preferred_element_type=jnp.float32)
                 + jnp.dot(h, wpmlp, preferred_element_type=jnp.float32) + bp)
        return jax.nn.sigmoid(logit)

    ps = score(_half(imf_r[...], pi_r[...]), imlp_r[...])
    ns = score(_half(nmf_r[...], pn_r[...]), nmlp_r[...])
    part = jnp.sum(jax.nn.softplus(ns - ps)) * (1.0 / _B)

    @pl.when(pl.program_id(0) == 0)
    def _():
        out_r[...] = jnp.zeros_like(out_r)

    out_r[...] += part


def _tc_loss(umf, imf, nmf, umlp, imlp, nmlp, pu, pi, pn,
             w1a, w1b, b1, w2, b2, w3, b3, wpmf, wpmlp, bp, *,
             interpret=False):
    bb = 2048
    nb = _B // bb

    def fixed(shape):
        return pl.BlockSpec(shape, lambda i: (0, 0))

    def batched(d):
        return pl.BlockSpec((bb, d), lambda i: (i, 0))

    return pl.pallas_call(
        _tc_body,
        grid=(nb,),
        in_specs=[
            batched(_DMLP), batched(_DMLP), batched(_DMLP),
            batched(_DMLP), batched(_DMLP), batched(_DMLP),
            batched(1), batched(1), batched(1),
            fixed((_DMLP, _DMLP)), fixed((_DMLP, _DMLP)), fixed((1, _DMLP)),
            fixed((_DMLP, 64)), fixed((1, 64)),
            fixed((64, 32)), fixed((1, 32)),
            fixed((_DMF, 1)), fixed((32, 1)), fixed((1, 1)),
        ],
        out_specs=pl.BlockSpec((1, 1), lambda i: (0, 0)),
        out_shape=jax.ShapeDtypeStruct((1, 1), jnp.float32),
        compiler_params=pltpu.CompilerParams(
            dimension_semantics=("arbitrary",)),
        interpret=interpret,
    )(umf, imf, nmf, umlp, imlp, nmlp, pu, pi, pn,
      w1a, w1b, b1, w2, b2, w3, b3, wpmf, wpmlp, bp)


def _chunked(stack):
    nw = 32
    bpw = _B // nw
    return (stack.reshape(_NSETS, nw, bpw // _CH, _CH)
            .transpose(1, 0, 2, 3)
            .reshape(nw, _NSETS * (bpw // _CH), _CH))


def kernel(user, item, neg_item, mf_user, mf_item, mlp_user, mlp_item,
           W1, b1, W2, b2, W3, b3, Wp, bp):
    user = user.astype(jnp.int32)
    item = item.astype(jnp.int32)
    neg_item = neg_item.astype(jnp.int32)
    hu = (user >= _OFF).astype(jnp.int32)
    hi = (item >= _OFF).astype(jnp.int32)
    hn = (neg_item >= _OFF).astype(jnp.int32)
    idx_mlp = _chunked(jnp.stack([user, item, neg_item]))
    idx_mf = _chunked(jnp.stack([user - hu * _OFF, item - hi * _OFF,
                                 neg_item - hn * _OFF]))
    pu = hu.reshape(_B, 1)
    pi = hi.reshape(_B, 1)
    pn = hn.reshape(_B, 1)

    umlp, imlp, nmlp = _make_gather(mlp_user.shape[0], mlp_item.shape[0])(
        idx_mlp, mlp_user, mlp_item)

    mf_user2, mf_item2 = _repack(mf_user.T, mf_item.T)
    umf, imf, nmf = _make_gather(mf_user2.shape[0], mf_item2.shape[0])(
        idx_mf, mf_user2, mf_item2)

    w1a = W1[:_DMLP]
    w1b = W1[_DMLP:]
    out = _tc_loss(
        umf, imf, nmf, umlp, imlp, nmlp, pu, pi, pn,
        w1a, w1b, b1.reshape(1, _DMLP), W2, b2.reshape(1, 64),
        W3, b3.reshape(1, 32), Wp[:_DMF], Wp[_DMF:], bp.reshape(1, 1))
    return out[0, 0]


# speedup vs baseline: 1.5877x; 1.1280x over previous
import functools

import jax
import jax.numpy as jnp
from jax import lax
from jax.experimental import pallas as pl
from jax.experimental.pallas import tpu as pltpu
from jax.experimental.pallas import tpu_sc as plsc

_B = 16384
_DMF = 64
_DMLP = 128
_CH = 128
_NSETS = 3
_DEPTH = 4
_LAG_W = 2


@functools.lru_cache(maxsize=None)
def _make_gather(rows_a, rows_b):
    info = plsc.get_sparse_core_info()
    nc, ns = info.num_cores, info.num_subcores
    nw = nc * ns
    bpw = _B // nw
    nch = bpw // _CH
    nk = _NSETS * nch
    mesh = plsc.VectorSubcoreMesh(core_axis_name="c", subcore_axis_name="s")

    @functools.partial(
        pl.kernel,
        mesh=mesh,
        out_type=[jax.ShapeDtypeStruct((_B, _DMLP), jnp.float32)
                  for _ in range(_NSETS)],
        scratch_types=[
            pltpu.VMEM((nk, _CH), jnp.int32),
        ] + [pltpu.VMEM((_CH, _DMLP), jnp.float32) for _ in range(_DEPTH)] + [
            pltpu.SemaphoreType.DMA,
            pltpu.SemaphoreType.DMA,
        ],
    )
    def gather(idx_all, table_a, table_b, o_u, o_i, o_n,
               idx_v, *bufs_and_sems):
        bufs = bufs_and_sems[:_DEPTH]
        sem_g, sem_w = bufs_and_sems[_DEPTH], bufs_and_sems[_DEPTH + 1]
        tables = [table_a, table_b, table_b]
        outs = [o_u, o_i, o_n]
        wid = lax.axis_index("s") * nc + lax.axis_index("c")
        base = wid * bpw
        pltpu.sync_copy(idx_all.at[wid], idx_v)

        gcps = [None] * nk
        wcps = [None] * nk

        def issue_write(m):
            s, j = divmod(m, nch)
            gcps[m].wait()
            wcps[m] = pltpu.async_copy(
                bufs[m % _DEPTH],
                outs[s].at[pl.ds(base + j * _CH, _CH)], sem_w)

        for k in range(nk):
            if k >= _LAG_W:
                issue_write(k - _LAG_W)
            if k >= _DEPTH:
                wcps[k - _DEPTH].wait()
            s = k // nch
            gcps[k] = pltpu.async_copy(
                tables[s].at[idx_v.at[k]], bufs[k % _DEPTH], sem_g)
        for m in range(nk - _LAG_W, nk):
            issue_write(m)
        for m in range(nk - _DEPTH, nk):
            wcps[m].wait()

    return gather


_NI = 100000
_RB = 2048
_RNB = 25
_OFF = _RB * _RNB
_RNB_IN = (_NI - _OFF) // _RB
_TAIL0 = _OFF + _RNB_IN * _RB


def _repack_body(au_r, bu_r, tu_r, ai_r, bi_r, ti_r, ou_r, oi_r):
    i = pl.program_id(0)
    ou_r[:, :_DMF] = au_r[...].T
    oi_r[:, :_DMF] = ai_r[...].T

    @pl.when(i < _RNB_IN)
    def _():
        ou_r[:, _DMF:] = bu_r[...].T
        oi_r[:, _DMF:] = bi_r[...].T

    @pl.when(i >= _RNB_IN)
    def _():
        ou_r[:, _DMF:] = tu_r[...].T
        oi_r[:, _DMF:] = ti_r[...].T


def _repack(mf_user_t, mf_item_t):
    ntail = _RNB - _RNB_IN
    tw = ntail * _RB

    def tail(t):
        return jnp.pad(t[:, _TAIL0:], ((0, 0), (0, tw - (_NI - _TAIL0))))

    a_map = lambda i: (0, i)
    b_map = lambda i: (0, jnp.minimum(i + _RNB, _RNB + _RNB_IN - 1))
    t_map = lambda i: (0, jnp.clip(i - _RNB_IN, 0, ntail - 1))
    return pl.pallas_call(
        _repack_body,
        grid=(_RNB,),
        in_specs=[pl.BlockSpec((_DMF, _RB), a_map),
                  pl.BlockSpec((_DMF, _RB), b_map),
                  pl.BlockSpec((_DMF, _RB), t_map),
                  pl.BlockSpec((_DMF, _RB), a_map),
                  pl.BlockSpec((_DMF, _RB), b_map),
                  pl.BlockSpec((_DMF, _RB), t_map)],
        out_specs=[pl.BlockSpec((_RB, _DMLP), lambda i: (i, 0)),
                   pl.BlockSpec((_RB, _DMLP), lambda i: (i, 0))],
        out_shape=[jax.ShapeDtypeStruct((_OFF, _DMLP), jnp.float32),
                   jax.ShapeDtypeStruct((_OFF, _DMLP), jnp.float32)],
        compiler_params=pltpu.CompilerParams(
            dimension_semantics=("arbitrary",)),
    )(mf_user_t, mf_user_t, tail(mf_user_t),
      mf_item_t, mf_item_t, tail(mf_item_t))


def _half(wide, par):
    return jnp.where(par == 1.0, wide[:, _DMF:], wide[:, :_DMF])


def _tc_body(umf_r, imf_r, nmf_r, umlp_r, imlp_r, nmlp_r,
             par_r,
             w1a_r, w1b_r, b1_r, w2_r, b2_r, w3_r, b3_r,
             wpmf_r, wpmlp_r, bp_r, out_r):
    pt = par_r[...].T
    w1a = w1a_r[...]
    w1b = w1b_r[...]
    b1 = b1_r[...]
    w2 = w2_r[...]
    b2 = b2_r[...]
    w3 = w3_r[...]
    b3 = b3_r[...]
    wpmf = wpmf_r[...]
    wpmlp = wpmlp_r[...]
    bp = bp_r[...]
    umf = _half(umf_r[...], pt[:, 0:1])
    umlp = umlp_r[...]
    u1 = jnp.dot(umlp, w1a, preferred_element_type=jnp.float32)

    def score(imf, imlp):
        h = jnp.maximum(
            u1 + jnp.dot(imlp, w1b, preferred_element_type=jnp.float32) + b1,
            0.0)
        h = jnp.maximum(
            jnp.dot(h, w2, preferred_element_type=jnp.float32) + b2, 0.0)
        h = jnp.maximum(
            jnp.dot(h, w3, preferred_element_type=jnp.float32) + b3, 0.0)
        logit = (jnp.dot(umf * imf, wpmf, preferred_element_type=jnp.float32)
                 + jnp.dot(h, wpmlp, preferred_element_type=jnp.float32) + bp)
        return jax.nn.sigmoid(logit)

    ps = score(_half(imf_r[...], pt[:, 1:2]), imlp_r[...])
    ns = score(_half(nmf_r[...], pt[:, 2:3]), nmlp_r[...])
    part = jnp.sum(jax.nn.softplus(ns - ps)) * (1.0 / _B)

    @pl.when(pl.program_id(0) == 0)
    def _():
        out_r[...] = jnp.zeros_like(out_r)

    out_r[...] += part


def _tc_loss(umf, imf, nmf, umlp, imlp, nmlp, par3,
             w1a, w1b, b1, w2, b2, w3, b3, wpmf, wpmlp, bp, *,
             interpret=False):
    bb = 2048
    nb = _B // bb

    def fixed(shape):
        return pl.BlockSpec(shape, lambda i: (0, 0))

    def batched(d):
        return pl.BlockSpec((bb, d), lambda i: (i, 0))

    return pl.pallas_call(
        _tc_body,
        grid=(nb,),
        in_specs=[
            batched(_DMLP), batched(_DMLP), batched(_DMLP),
            batched(_DMLP), batched(_DMLP), batched(_DMLP),
            pl.BlockSpec((3, bb), lambda i: (0, i)),
            fixed((_DMLP, _DMLP)), fixed((_DMLP, _DMLP)), fixed((1, _DMLP)),
            fixed((_DMLP, 64)), fixed((1, 64)),
            fixed((64, 32)), fixed((1, 32)),
            fixed((_DMF, 1)), fixed((32, 1)), fixed((1, 1)),
        ],
        out_specs=pl.BlockSpec((1, 1), lambda i: (0, 0)),
        out_shape=jax.ShapeDtypeStruct((1, 1), jnp.float32),
        compiler_params=pltpu.CompilerParams(
            dimension_semantics=("arbitrary",)),
        interpret=interpret,
    )(umf, imf, nmf, umlp, imlp, nmlp, par3,
      w1a, w1b, b1, w2, b2, w3, b3, wpmf, wpmlp, bp)


def _chunked(stack):
    nw = 32
    bpw = _B // nw
    return (stack.reshape(_NSETS, nw, bpw // _CH, _CH)
            .transpose(1, 0, 2, 3)
            .reshape(nw, _NSETS * (bpw // _CH), _CH))


def kernel(user, item, neg_item, mf_user, mf_item, mlp_user, mlp_item,
           W1, b1, W2, b2, W3, b3, Wp, bp):
    user = user.astype(jnp.int32)
    item = item.astype(jnp.int32)
    neg_item = neg_item.astype(jnp.int32)
    hu = (user >= _OFF).astype(jnp.int32)
    hi = (item >= _OFF).astype(jnp.int32)
    hn = (neg_item >= _OFF).astype(jnp.int32)
    idx_mlp = _chunked(jnp.stack([user, item, neg_item]))
    idx_mf = _chunked(jnp.stack([user - hu * _OFF, item - hi * _OFF,
                                 neg_item - hn * _OFF]))
    par3 = jnp.stack([hu, hi, hn]).astype(jnp.float32)

    umlp, imlp, nmlp = _make_gather(mlp_user.shape[0], mlp_item.shape[0])(
        idx_mlp, mlp_user, mlp_item)

    mf_user2, mf_item2 = _repack(mf_user.T, mf_item.T)
    umf, imf, nmf = _make_gather(mf_user2.shape[0], mf_item2.shape[0])(
        idx_mf, mf_user2, mf_item2)

    w1a = W1[:_DMLP]
    w1b = W1[_DMLP:]
    out = _tc_loss(
        umf, imf, nmf, umlp, imlp, nmlp, par3,
        w1a, w1b, b1.reshape(1, _DMLP), W2, b2.reshape(1, 64),
        W3, b3.reshape(1, 32), Wp[:_DMF], Wp[_DMF:], bp.reshape(1, 1))
    return out[0, 0]


# speedup vs baseline: 1.6273x; 1.0249x over previous
import functools

import jax
import jax.numpy as jnp
from jax import lax
from jax.experimental import pallas as pl
from jax.experimental.pallas import tpu as pltpu
from jax.experimental.pallas import tpu_sc as plsc

_B = 16384
_DMF = 64
_DMLP = 128
_CH = 128
_NSETS = 3
_DEPTH = 4
_LAG_W = 2


@functools.lru_cache(maxsize=None)
def _make_gather(rows_a, rows_b):
    info = plsc.get_sparse_core_info()
    nc, ns = info.num_cores, info.num_subcores
    nw = nc * ns
    bpw = _B // nw
    nch = bpw // _CH
    nk = _NSETS * nch
    mesh = plsc.VectorSubcoreMesh(core_axis_name="c", subcore_axis_name="s")

    @functools.partial(
        pl.kernel,
        mesh=mesh,
        out_type=[jax.ShapeDtypeStruct((_B, _DMLP), jnp.float32)
                  for _ in range(_NSETS)],
        scratch_types=[
            pltpu.VMEM((nk, _CH), jnp.int32),
        ] + [pltpu.VMEM((_CH, _DMLP), jnp.float32) for _ in range(_DEPTH)] + [
            pltpu.SemaphoreType.DMA,
            pltpu.SemaphoreType.DMA,
        ],
    )
    def gather(idx_all, table_a, table_b, o_u, o_i, o_n,
               idx_v, *bufs_and_sems):
        bufs = bufs_and_sems[:_DEPTH]
        sem_g, sem_w = bufs_and_sems[_DEPTH], bufs_and_sems[_DEPTH + 1]
        tables = [table_a, table_b, table_b]
        outs = [o_u, o_i, o_n]
        wid = lax.axis_index("s") * nc + lax.axis_index("c")
        base = wid * bpw
        pltpu.sync_copy(idx_all.at[wid], idx_v)

        gcps = [None] * nk
        wcps = [None] * nk

        def issue_write(m):
            s, j = divmod(m, nch)
            gcps[m].wait()
            wcps[m] = pltpu.async_copy(
                bufs[m % _DEPTH],
                outs[s].at[pl.ds(base + j * _CH, _CH)], sem_w)

        for k in range(nk):
            if k >= _LAG_W:
                issue_write(k - _LAG_W)
            if k >= _DEPTH:
                wcps[k - _DEPTH].wait()
            s = k // nch
            gcps[k] = pltpu.async_copy(
                tables[s].at[idx_v.at[k]], bufs[k % _DEPTH], sem_g)
        for m in range(nk - _LAG_W, nk):
            issue_write(m)
        for m in range(nk - _DEPTH, nk):
            wcps[m].wait()

    return gather


_NI = 100000
_RB = 4096
_RNB = 13
_OFF = _RB * _RNB
_RNB_IN = (_NI - _OFF) // _RB
_TAIL0 = _OFF + _RNB_IN * _RB


def _repack_body(au_r, bu_r, tu_r, ai_r, bi_r, ti_r, ou_r, oi_r):
    i = pl.program_id(0)
    ou_r[:, :_DMF] = au_r[...].T
    oi_r[:, :_DMF] = ai_r[...].T

    @pl.when(i < _RNB_IN)
    def _():
        ou_r[:, _DMF:] = bu_r[...].T
        oi_r[:, _DMF:] = bi_r[...].T

    @pl.when(i >= _RNB_IN)
    def _():
        ou_r[:, _DMF:] = tu_r[...].T
        oi_r[:, _DMF:] = ti_r[...].T


def _repack(mf_user_t, mf_item_t):
    ntail = _RNB - _RNB_IN
    tw = ntail * _RB

    def tail(t):
        return jnp.pad(t[:, _TAIL0:], ((0, 0), (0, tw - (_NI - _TAIL0))))

    a_map = lambda i: (0, i)
    b_map = lambda i: (0, jnp.minimum(i + _RNB, _RNB + _RNB_IN - 1))
    t_map = lambda i: (0, jnp.clip(i - _RNB_IN, 0, ntail - 1))
    return pl.pallas_call(
        _repack_body,
        grid=(_RNB,),
        in_specs=[pl.BlockSpec((_DMF, _RB), a_map),
                  pl.BlockSpec((_DMF, _RB), b_map),
                  pl.BlockSpec((_DMF, _RB), t_map),
                  pl.BlockSpec((_DMF, _RB), a_map),
                  pl.BlockSpec((_DMF, _RB), b_map),
                  pl.BlockSpec((_DMF, _RB), t_map)],
        out_specs=[pl.BlockSpec((_RB, _DMLP), lambda i: (i, 0)),
                   pl.BlockSpec((_RB, _DMLP), lambda i: (i, 0))],
        out_shape=[jax.ShapeDtypeStruct((_OFF, _DMLP), jnp.float32),
                   jax.ShapeDtypeStruct((_OFF, _DMLP), jnp.float32)],
        compiler_params=pltpu.CompilerParams(
            dimension_semantics=("arbitrary",)),
    )(mf_user_t, mf_user_t, tail(mf_user_t),
      mf_item_t, mf_item_t, tail(mf_item_t))


def _half(wide, par):
    return jnp.where(par == 1.0, wide[:, _DMF:], wide[:, :_DMF])


def _tc_body(umf_r, imf_r, nmf_r, umlp_r, imlp_r, nmlp_r,
             par_r,
             w1a_r, w1b_r, b1_r, w2_r, b2_r, w3_r, b3_r,
             wpmf_r, wpmlp_r, bp_r, out_r):
    pt = par_r[...].T
    w1a = w1a_r[...]
    w1b = w1b_r[...]
    b1 = b1_r[...]
    w2 = w2_r[...]
    b2 = b2_r[...]
    w3 = w3_r[...]
    b3 = b3_r[...]
    wpmf = wpmf_r[...]
    wpmlp = wpmlp_r[...]
    bp = bp_r[...]
    umf = _half(umf_r[...], pt[:, 0:1])
    umlp = umlp_r[...]
    u1 = jnp.dot(umlp, w1a, preferred_element_type=jnp.float32)

    def score(imf, imlp):
        h = jnp.maximum(
            u1 + jnp.dot(imlp, w1b, preferred_element_type=jnp.float32) + b1,
            0.0)
        h = jnp.maximum(
            jnp.dot(h, w2, preferred_element_type=jnp.float32) + b2, 0.0)
        h = jnp.maximum(
            jnp.dot(h, w3, preferred_element_type=jnp.float32) + b3, 0.0)
        logit = (jnp.dot(umf * imf, wpmf, preferred_element_type=jnp.float32)
                 + jnp.dot(h, wpmlp, preferred_element_type=jnp.float32) + bp)
        return jax.nn.sigmoid(logit)

    ps = score(_half(imf_r[...], pt[:, 1:2]), imlp_r[...])
    ns = score(_half(nmf_r[...], pt[:, 2:3]), nmlp_r[...])
    part = jnp.sum(jax.nn.softplus(ns - ps)) * (1.0 / _B)

    @pl.when(pl.program_id(0) == 0)
    def _():
        out_r[...] = jnp.zeros_like(out_r)

    out_r[...] += part


def _tc_loss(umf, imf, nmf, umlp, imlp, nmlp, par3,
             w1a, w1b, b1, w2, b2, w3, b3, wpmf, wpmlp, bp, *,
             interpret=False):
    bb = 2048
    nb = _B // bb

    def fixed(shape):
        return pl.BlockSpec(shape, lambda i: (0, 0))

    def batched(d):
        return pl.BlockSpec((bb, d), lambda i: (i, 0))

    return pl.pallas_call(
        _tc_body,
        grid=(nb,),
        in_specs=[
            batched(_DMLP), batched(_DMLP), batched(_DMLP),
            batched(_DMLP), batched(_DMLP), batched(_DMLP),
            pl.BlockSpec((3, bb), lambda i: (0, i)),
            fixed((_DMLP, _DMLP)), fixed((_DMLP, _DMLP)), fixed((1, _DMLP)),
            fixed((_DMLP, 64)), fixed((1, 64)),
            fixed((64, 32)), fixed((1, 32)),
            fixed((_DMF, 1)), fixed((32, 1)), fixed((1, 1)),
        ],
        out_specs=pl.BlockSpec((1, 1), lambda i: (0, 0)),
        out_shape=jax.ShapeDtypeStruct((1, 1), jnp.float32),
        compiler_params=pltpu.CompilerParams(
            dimension_semantics=("arbitrary",)),
        interpret=interpret,
    )(umf, imf, nmf, umlp, imlp, nmlp, par3,
      w1a, w1b, b1, w2, b2, w3, b3, wpmf, wpmlp, bp)


def _chunked(stack):
    nw = 32
    bpw = _B // nw
    return (stack.reshape(_NSETS, nw, bpw // _CH, _CH)
            .transpose(1, 0, 2, 3)
            .reshape(nw, _NSETS * (bpw // _CH), _CH))


def kernel(user, item, neg_item, mf_user, mf_item, mlp_user, mlp_item,
           W1, b1, W2, b2, W3, b3, Wp, bp):
    user = user.astype(jnp.int32)
    item = item.astype(jnp.int32)
    neg_item = neg_item.astype(jnp.int32)
    hu = (user >= _OFF).astype(jnp.int32)
    hi = (item >= _OFF).astype(jnp.int32)
    hn = (neg_item >= _OFF).astype(jnp.int32)
    idx_mlp = _chunked(jnp.stack([user, item, neg_item]))
    idx_mf = _chunked(jnp.stack([user - hu * _OFF, item - hi * _OFF,
                                 neg_item - hn * _OFF]))
    par3 = jnp.stack([hu, hi, hn]).astype(jnp.float32)

    umlp, imlp, nmlp = _make_gather(mlp_user.shape[0], mlp_item.shape[0])(
        idx_mlp, mlp_user, mlp_item)

    mf_user2, mf_item2 = _repack(mf_user.T, mf_item.T)
    umf, imf, nmf = _make_gather(mf_user2.shape[0], mf_item2.shape[0])(
        idx_mf, mf_user2, mf_item2)

    w1a = W1[:_DMLP]
    w1b = W1[_DMLP:]
    out = _tc_loss(
        umf, imf, nmf, umlp, imlp, nmlp, par3,
        w1a, w1b, b1.reshape(1, _DMLP), W2, b2.reshape(1, 64),
        W3, b3.reshape(1, 32), Wp[:_DMF], Wp[_DMF:], bp.reshape(1, 1))
    return out[0, 0]
